# baseline (speedup 1.0000x reference)
import functools

import jax
import jax.numpy as jnp
import numpy as np
from jax import lax
from jax.experimental import pallas as pl
from jax.experimental.pallas import tpu as pltpu
from jax.experimental.pallas import tpu_sc as plsc

_C = 3
_HR = 512
_LR = 128
_PS = 16
_NH = 32
_N = 1024
_POOL = 4
_DESC = 48
_EMB = 64
_TOPK = 32
_FLAT = _C * _PS * _PS

_NWRK = 32


def _np_gauss1d():
    x = np.arange(5, dtype=np.float64) - 2.0
    k1 = np.exp(-(x ** 2) / 2.0)
    return (k1 / k1.sum()).astype(np.float32)


def _np_blur_mat():
    k1 = _np_gauss1d()
    G = np.zeros((_HR, _HR), np.float32)
    for t in range(5):
        off = t - 2
        for i in range(_HR):
            j = i + off
            if 0 <= j < _HR:
                G[i, j] += k1[t]
    return G


def _np_keys_cubic(t, a=-0.5):
    t = np.abs(t)
    return np.where(t <= 1, (a + 2) * t ** 3 - (a + 3) * t ** 2 + 1,
                    np.where(t < 2, a * (t ** 3 - 5 * t ** 2 + 8 * t - 4), 0.0))


def _np_resize_mat():
    scale = _HR / _LR
    x = (np.arange(_HR) + 0.5) / scale - 0.5
    j = np.arange(_LR)
    w = _np_keys_cubic(x[:, None] - j[None, :])
    return (w / w.sum(axis=1, keepdims=True)).astype(np.float32)


def _np_pool_mat():
    A = np.zeros((_HR // 4, _HR), np.float32)
    for i in range(_HR // 4):
        A[i, 4 * i:4 * i + 4] = 0.25
    return A


_G_NP = _np_blur_mat()
_R_NP = _np_resize_mat()
_RB_NP = (_G_NP @ _R_NP).astype(np.float32)
_AH_NP = _np_pool_mat()

_PREC = lax.Precision.DEFAULT


def _dot(a, b):
    return lax.dot_general(a, b, (((1,), (0,)), ((), ())),
                           precision=_PREC, preferred_element_type=jnp.float32)


def _dot_t(a, b):
    return lax.dot_general(a, b, (((1,), (1,)), ((), ())),
                           precision=_PREC, preferred_element_type=jnp.float32)


def _prep_body(xhr_ref, xlr_ref, g_ref, r_ref, rb_ref, ah_ref,
               base_ref, hf_ref, hfp_ref, bhfp_ref):
    G = g_ref[...]
    R = r_ref[...]
    RB = rb_ref[...]
    AH = ah_ref[...]
    for c in range(_C):
        xc = xhr_ref[c]
        xl = xlr_ref[c]
        base = _dot_t(_dot(R, xl), R)
        bb = _dot_t(_dot(RB, xl), RB)
        bhf = base - bb
        blur = _dot_t(_dot(G, xc), G)
        hf = xc - blur
        base_ref[c] = base
        hf_ref[c] = hf
        hfp_ref[c] = _dot_t(_dot(AH, hf), AH)
        bhfp_ref[c] = _dot_t(_dot(AH, bhf), AH)


def _prep(x_hr, x_lr):
    return pl.pallas_call(
        _prep_body,
        out_shape=(
            jax.ShapeDtypeStruct((_C, _HR, _HR), jnp.float32),
            jax.ShapeDtypeStruct((_C, _HR, _HR), jnp.float32),
            jax.ShapeDtypeStruct((_C, _LR, _LR), jnp.float32),
            jax.ShapeDtypeStruct((_C, _LR, _LR), jnp.float32),
        ),
    )(x_hr, x_lr, jnp.asarray(_G_NP), jnp.asarray(_R_NP),
      jnp.asarray(_RB_NP), jnp.asarray(_AH_NP))


def _sc_patchify(hf2d, hfp2d, bhfp2d):
    mesh = plsc.VectorSubcoreMesh(core_axis_name="c", subcore_axis_name="s")

    @functools.partial(
        pl.kernel, mesh=mesh,
        out_type=(jax.ShapeDtypeStruct((_N * _N,), jnp.float32),
                  jax.ShapeDtypeStruct((_N * _DESC,), jnp.float32),
                  jax.ShapeDtypeStruct((_N * _DESC,), jnp.float32)),
        compiler_params=pltpu.CompilerParams(needs_layout_passes=False),
        scratch_types=[
            pltpu.VMEM((48, 512), jnp.float32),
            pltpu.VMEM((24, 128), jnp.float32),
            pltpu.VMEM((24, 128), jnp.float32),
            pltpu.VMEM((32 * _N,), jnp.float32),
            pltpu.VMEM((32 * _DESC,), jnp.float32),
            pltpu.VMEM((32 * _DESC,), jnp.float32),
            pltpu.SemaphoreType.DMA,
        ],
    )
    def k(hf_hbm, hfp_hbm, bhfp_hbm, hff_hbm, kp_hbm, qp_hbm,
          bandv, pbandv, qbandv, hstage, kstage, qstage, sem):
        wid = lax.axis_index("s") * 2 + lax.axis_index("c")
        iota = lax.iota(jnp.int32, 16)

        cps = []
        for c in range(_C):
            cps.append(pltpu.make_async_copy(
                hf_hbm.at[pl.ds(c * _HR + 16 * wid, 16), :],
                bandv.at[pl.ds(c * 16, 16), :], sem))
            cps.append(pltpu.make_async_copy(
                hfp_hbm.at[pl.ds(c * _LR + 8 * (wid // 2), 8), :],
                pbandv.at[pl.ds(c * 8, 8), :], sem))
            cps.append(pltpu.make_async_copy(
                bhfp_hbm.at[pl.ds(c * _LR + 8 * (wid // 2), 8), :],
                qbandv.at[pl.ds(c * 8, 8), :], sem))
        for cp in cps:
            cp.start()
        for cp in cps:
            cp.wait()

        @pl.loop(0, 48)
        def _(ci):
            for tw in range(32):
                hstage[pl.ds(tw * _N + 16 * ci, 16)] = (
                    bandv[ci, pl.ds(16 * tw, 16)])

        off = 4 * (wid % 2)
        for g in range(_C):
            rvec = g * 8 + off + iota // 4
            for tw in range(32):
                cvec = 4 * tw + (iota % 4)
                kstage[pl.ds(tw * _DESC + 16 * g, 16)] = (
                    plsc.load_gather(pbandv, [rvec, cvec]))
                qstage[pl.ds(tw * _DESC + 16 * g, 16)] = (
                    plsc.load_gather(qbandv, [rvec, cvec]))

        ocps = [
            pltpu.make_async_copy(
                hstage, hff_hbm.at[pl.ds(wid * 32 * _N, 32 * _N)], sem),
            pltpu.make_async_copy(
                kstage, kp_hbm.at[pl.ds(wid * 32 * _DESC, 32 * _DESC)], sem),
            pltpu.make_async_copy(
                qstage, qp_hbm.at[pl.ds(wid * 32 * _DESC, 32 * _DESC)], sem),
        ]
        for cp in ocps:
            cp.start()
        for cp in ocps:
            cp.wait()

    return k(hf2d, hfp2d, bhfp2d)


def _sc_fold(resc, base2d):
    mesh = plsc.VectorSubcoreMesh(core_axis_name="c", subcore_axis_name="s")

    @functools.partial(
        pl.kernel, mesh=mesh,
        out_type=jax.ShapeDtypeStruct((_C * _HR, _HR), jnp.float32),
        compiler_params=pltpu.CompilerParams(needs_layout_passes=False),
        scratch_types=[
            pltpu.VMEM((32, _FLAT), jnp.float32),
            pltpu.VMEM((48, 512), jnp.float32),
            pltpu.VMEM((48, 512), jnp.float32),
            pltpu.SemaphoreType.DMA,
        ],
    )
    def k(resc_hbm, base_hbm, out_hbm, rv, bv, ov, sem):
        wid = lax.axis_index("s") * 2 + lax.axis_index("c")
        cps = [pltpu.make_async_copy(
            resc_hbm.at[pl.ds(wid * 32, 32), :], rv, sem)]
        for c in range(_C):
            cps.append(pltpu.make_async_copy(
                base_hbm.at[pl.ds(c * _HR + 16 * wid, 16), :],
                bv.at[pl.ds(c * 16, 16), :], sem))
        for cp in cps:
            cp.start()
        for cp in cps:
            cp.wait()

        @pl.loop(0, 48)
        def _(ci):
            for tw in range(32):
                ov[ci, pl.ds(16 * tw, 16)] = (
                    bv[ci, pl.ds(16 * tw, 16)] +
                    rv[tw, pl.ds(16 * ci, 16)])

        ocps = []
        for c in range(_C):
            ocps.append(pltpu.make_async_copy(
                ov.at[pl.ds(c * 16, 16), :],
                out_hbm.at[pl.ds(c * _HR + 16 * wid, 16), :], sem))
        for cp in ocps:
            cp.start()
        for cp in ocps:
            cp.wait()

    return k(resc, base2d)


def _sc_topk(attn_flat):
    rpw = _N // _NWRK
    RIL = 8

    mesh = plsc.VectorSubcoreMesh(core_axis_name="c", subcore_axis_name="s")

    @functools.partial(
        pl.kernel, mesh=mesh,
        out_type=(jax.ShapeDtypeStruct((_N * _TOPK,), jnp.float32),
                  jax.ShapeDtypeStruct((_N * _TOPK,), jnp.int32)),
        compiler_params=pltpu.CompilerParams(needs_layout_passes=False),
        scratch_types=[
            pltpu.VMEM((rpw * _N,), jnp.float32),
            pltpu.VMEM((rpw * _TOPK,), jnp.float32),
            pltpu.VMEM((rpw * _TOPK,), jnp.int32),
        ],
    )
    def k(attn_hbm, prior_hbm, idx_hbm, rows, pv, iv):
        wid = lax.axis_index("s") * 2 + lax.axis_index("c")
        pltpu.sync_copy(attn_hbm.at[pl.ds(wid * rpw * _N, rpw * _N)], rows)
        iota = lax.iota(jnp.int32, 16)

        def row_cm(rbase):
            cms = []
            for j in range(4):
                cm = rows[pl.ds(rbase + j * 256, 16)]
                for i in range(1, 16):
                    cm = jnp.maximum(cm, rows[pl.ds(rbase + j * 256 + i * 16, 16)])
                cms.append(cm)
            return cms

        def extract(rbase, cms):
            cm0, cm1, cm2, cm3 = cms
            t = jnp.maximum(jnp.maximum(cm0, cm1), jnp.maximum(cm2, cm3))
            M = jnp.max(t)
            Ms = jnp.full((16,), M)
            eqs = [cm == Ms for cm in cms]
            ps = [plsc.all_reduce_population_count(e) for e in eqs]
            fs = [plsc.all_reduce_ffs(e) for e in eqs]
            jsel = jnp.full((16,), 3, jnp.int32)
            lsel = fs[3]
            for j in (2, 1, 0):
                hit = ps[j] > 0
                jsel = jnp.where(hit, j, jsel)
                lsel = jnp.where(hit, fs[j], lsel)
            colbase = jsel * 256 + lsel
            cvals = plsc.load_gather(rows, [rbase + colbase + iota * 16])
            eqc = cvals == Ms
            fi = plsc.all_reduce_ffs(eqc)
            col = colbase + fi * 16
            lane0 = iota == 0
            neg1 = jnp.full((16,), -1.0)
            plsc.store_scatter(rows, [rbase + col], neg1, mask=lane0)
            cv2 = jnp.where(iota == fi, neg1, cvals)
            nms = jnp.full((16,), jnp.max(cv2))
            new = []
            for j, cm in enumerate(cms):
                new.append(jnp.where((jsel == j) & (iota == lsel), nms, cm))
            return col, Ms, new

        def grp_body(g, _):
            rlocs = [g * RIL + u for u in range(RIL)]
            rbases = [rl * _N for rl in rlocs]
            init = tuple(x for rb in rbases for x in row_cm(rb))

            lane0 = iota == 0

            def step(kk, carry):
                out = []
                for u in range(RIL):
                    col, Ms, ncms = extract(rbases[u], list(carry[4 * u:4 * u + 4]))
                    addr = jnp.full((16,), rlocs[u] * _TOPK, jnp.int32) + kk
                    plsc.store_scatter(pv, [addr], Ms, mask=lane0)
                    plsc.store_scatter(iv, [addr], col, mask=lane0)
                    out += ncms
                return tuple(out)

            lax.fori_loop(0, _TOPK, step, init)
            return 0

        lax.fori_loop(0, rpw // RIL, grp_body, 0)
        pltpu.sync_copy(pv, prior_hbm.at[pl.ds(wid * rpw * _TOPK, rpw * _TOPK)])
        pltpu.sync_copy(iv, idx_hbm.at[pl.ds(wid * rpw * _TOPK, rpw * _TOPK)])

    return k(attn_flat)


def _ktable_body(kp_ref, wk_ref, bk_ref, w1_ref, out_ref):
    kge = _dot(kp_ref[...], wk_ref[...]) + bk_ref[...]
    w1 = w1_ref[...]
    w1kd = w1[_EMB:2 * _EMB] - w1[2 * _EMB:3 * _EMB]
    out_ref[:, 0:_EMB] = kge
    out_ref[:, _EMB:2 * _EMB] = _dot(kge, w1kd)


def _ktable(kp_flat, Wk, bk, W1):
    return pl.pallas_call(
        _ktable_body,
        out_shape=jax.ShapeDtypeStruct((_N, 2 * _EMB), jnp.float32),
    )(kp_flat, Wk, bk.reshape(1, _EMB), W1)


def _sc_gather(table, idx_flat):
    bpw = (_N * _TOPK) // _NWRK
    rows_per_w = bpw // 128

    mesh = plsc.VectorSubcoreMesh(core_axis_name="c", subcore_axis_name="s")

    @functools.partial(
        pl.kernel, mesh=mesh,
        out_type=jax.ShapeDtypeStruct((_N * _TOPK, 2 * _EMB), jnp.float32),
        scratch_types=[
            pltpu.VMEM((bpw,), jnp.int32),
            pltpu.VMEM((128, 2 * _EMB), jnp.float32),
            pltpu.VMEM((128, 2 * _EMB), jnp.float32),
            pltpu.SemaphoreType.DMA,
            pltpu.SemaphoreType.DMA,
        ],
    )
    def k(table_hbm, idx_hbm, out_hbm, idx_v, buf0, buf1, sem0, sem1):
        wid = lax.axis_index("s") * 2 + lax.axis_index("c")
        pltpu.sync_copy(idx_hbm.at[pl.ds(wid * bpw, bpw)], idx_v)
        bufs = (buf0, buf1)
        sems = (sem0, sem1)

        def chunk(j):
            return table_hbm.at[idx_v.at[pl.ds(j * 128, 128)]]

        pltpu.async_copy(chunk(0), bufs[0], sems[0])
        for j in range(rows_per_w):
            if j + 1 < rows_per_w:
                pltpu.async_copy(chunk(j + 1), bufs[(j + 1) % 2],
                                 sems[(j + 1) % 2])
            pltpu.make_async_copy(chunk(j), bufs[j % 2], sems[j % 2]).wait()
            pltpu.sync_copy(bufs[j % 2],
                            out_hbm.at[pl.ds(wid * bpw + j * 128, 128)])

    return k(table, idx_flat)


def _sc_scatter(w_flat, idx_flat, zrows):
    rpw = _N // _NWRK
    vpw = rpw * _TOPK

    mesh = plsc.VectorSubcoreMesh(core_axis_name="c", subcore_axis_name="s")

    @functools.partial(
        pl.kernel, mesh=mesh,
        out_type=jax.ShapeDtypeStruct((_N * _N,), jnp.float32),
        compiler_params=pltpu.CompilerParams(needs_layout_passes=False),
        scratch_types=[
            pltpu.VMEM((vpw,), jnp.int32),
            pltpu.VMEM((vpw,), jnp.float32),
            pltpu.VMEM((rpw * _N,), jnp.float32),
        ],
    )
    def k(w_hbm, idx_hbm, z_hbm, s_hbm, idx_v, w_v, srow):
        wid = lax.axis_index("s") * 2 + lax.axis_index("c")
        pltpu.sync_copy(z_hbm, srow)
        pltpu.sync_copy(idx_hbm.at[pl.ds(wid * vpw, vpw)], idx_v)
        pltpu.sync_copy(w_hbm.at[pl.ds(wid * vpw, vpw)], w_v)
        for r in range(rpw):
            roff = jnp.full((16,), r * _N, jnp.int32)
            for h in range(_TOPK // 16):
                cols = idx_v[pl.ds(r * _TOPK + h * 16, 16)]
                vals = w_v[pl.ds(r * _TOPK + h * 16, 16)]
                plsc.store_scatter(srow, [cols + roff], vals)
        pltpu.sync_copy(srow, s_hbm.at[pl.ds(wid * rpw * _N, rpw * _N)])

    return k(w_flat, idx_flat, zrows)


def _mlp_body(qp_ref, kg_ref, prior_ref, wq_ref, bq_ref,
              w1_ref, b1_ref, w2_ref, b2_ref, w_ref):
    blk = w_ref.shape[0]
    prior = prior_ref[...]
    qe = _dot(qp_ref[...], wq_ref[...]) + bq_ref[...]
    kge = kg_ref[:, 0:_EMB]
    Bk = kg_ref[:, _EMB:2 * _EMB]
    w1 = w1_ref[...]
    w1q = w1[0:_EMB]
    w1d = w1[2 * _EMB:3 * _EMB]
    w1m = w1[3 * _EMB:4 * _EMB]
    w1p = w1[4 * _EMB:4 * _EMB + 1]
    A = _dot(qe, w1q + w1d) + b1_ref[...]
    kge3 = kge.reshape(blk, _TOPK, _EMB)
    qk = (qe.reshape(blk, 1, _EMB) * kge3).reshape(blk * _TOPK, _EMB)
    Cm = _dot(qk, w1m)
    h = (Bk + Cm).reshape(blk, _TOPK, _EMB)
    h = h + A.reshape(blk, 1, _EMB)
    h = h + prior.reshape(blk, _TOPK, 1) * w1p.reshape(1, 1, _EMB)
    h = jnp.maximum(h, 0.0)
    res = jnp.sum(h * w2_ref[...].reshape(1, 1, _EMB), axis=-1) + b2_ref[0, 0]
    logits = jnp.log(jnp.maximum(prior, 1e-8)) + res
    m = jnp.max(logits, axis=-1, keepdims=True)
    e = jnp.exp(logits - m)
    w_ref[...] = e / jnp.sum(e, axis=-1, keepdims=True)


def _mlp(qp, kg, prior, Wq, bq, W1, b1, W2, b2):
    blk = 256
    grid = _N // blk
    return pl.pallas_call(
        _mlp_body,
        grid=(grid,),
        in_specs=[
            pl.BlockSpec((blk, _DESC), lambda i: (i, 0)),
            pl.BlockSpec((blk * _TOPK, 2 * _EMB), lambda i: (i, 0)),
            pl.BlockSpec((blk, _TOPK), lambda i: (i, 0)),
            pl.BlockSpec((_DESC, _EMB), lambda i: (0, 0)),
            pl.BlockSpec((1, _EMB), lambda i: (0, 0)),
            pl.BlockSpec((4 * _EMB + 1, _EMB), lambda i: (0, 0)),
            pl.BlockSpec((1, _EMB), lambda i: (0, 0)),
            pl.BlockSpec((_EMB, 1), lambda i: (0, 0)),
            pl.BlockSpec((1, 1), lambda i: (0, 0)),
        ],
        out_specs=pl.BlockSpec((blk, _TOPK), lambda i: (i, 0)),
        out_shape=jax.ShapeDtypeStruct((_N, _TOPK), jnp.float32),
    )(qp, kg, prior, Wq, bq.reshape(1, _EMB),
      W1, b1.reshape(1, _EMB), W2, b2.reshape(1, 1))


def _combine_body(s_ref, hf_ref, out_ref):
    blk = out_ref.shape[0]
    s = s_ref[...].reshape(blk, _N)
    hf = hf_ref[...].reshape(_N, _N)
    out_ref[...] = _dot(s, hf)[:, 0:_FLAT]


def _combine(S_flat, hf_flat):
    blk = 256
    return pl.pallas_call(
        _combine_body,
        grid=(_N // blk,),
        in_specs=[
            pl.BlockSpec((blk * _N,), lambda i: (i,)),
            pl.BlockSpec((_N * _N,), lambda i: (0,)),
        ],
        out_specs=pl.BlockSpec((blk, _FLAT), lambda i: (i, 0)),
        out_shape=jax.ShapeDtypeStruct((_N, _FLAT), jnp.float32),
    )(S_flat, hf_flat)


def _img_to_flat(img):
    x = img.reshape(_C, _NH, _PS, _NH, _PS)
    return x.transpose(1, 3, 0, 2, 4).reshape(_N, _FLAT)


def _pooled_to_flat(img):
    x = img.reshape(_C, _NH, _POOL, _NH, _POOL)
    return x.transpose(1, 3, 0, 2, 4).reshape(_N, _DESC)


def _flat_to_img(pf):
    x = pf.reshape(_NH, _NH, _C, _PS, _PS)
    return x.transpose(2, 0, 3, 1, 4).reshape(1, _C, _HR, _HR)


def kernel(x_hr, x_lr_inpainted, attn_map, Wk, bk, Wq, bq, W1, b1, W2, b2):
    base_img, hf_img, hfp, bhfp = _prep(x_hr[0], x_lr_inpainted[0])

    hf_flat, kp_flat, qp_flat = _sc_patchify(
        hf_img.reshape(_C * _HR, _HR),
        hfp.reshape(_C * _LR, _LR),
        bhfp.reshape(_C * _LR, _LR))

    pr_flat, ix_flat = _sc_topk(attn_map.reshape(-1))

    ktab = _ktable(kp_flat.reshape(_N, _DESC), Wk, bk, W1)
    kg = _sc_gather(ktab, ix_flat)
    w = _mlp(qp_flat.reshape(_N, _DESC), kg, pr_flat.reshape(_N, _TOPK),
             Wq, bq, W1, b1, W2, b2)

    S_flat = _sc_scatter(w.reshape(-1), ix_flat,
                         jnp.zeros((_N * _TOPK,), jnp.float32))
    resc = _combine(S_flat, hf_flat)
    final2d = _sc_fold(resc, base_img.reshape(_C * _HR, _HR))
    return final2d.reshape(1, _C, _HR, _HR)


# speedup vs baseline: 1.1270x; 1.1270x over previous
import functools

import jax
import jax.numpy as jnp
import numpy as np
from jax import lax
from jax.experimental import pallas as pl
from jax.experimental.pallas import tpu as pltpu
from jax.experimental.pallas import tpu_sc as plsc

_C = 3
_HR = 512
_LR = 128
_PS = 16
_NH = 32
_N = 1024
_POOL = 4
_DESC = 48
_EMB = 64
_TOPK = 32
_FLAT = _C * _PS * _PS

_NWRK = 32


def _np_gauss1d():
    x = np.arange(5, dtype=np.float64) - 2.0
    k1 = np.exp(-(x ** 2) / 2.0)
    return (k1 / k1.sum()).astype(np.float32)


def _np_blur_mat():
    k1 = _np_gauss1d()
    G = np.zeros((_HR, _HR), np.float32)
    for t in range(5):
        off = t - 2
        for i in range(_HR):
            j = i + off
            if 0 <= j < _HR:
                G[i, j] += k1[t]
    return G


def _np_keys_cubic(t, a=-0.5):
    t = np.abs(t)
    return np.where(t <= 1, (a + 2) * t ** 3 - (a + 3) * t ** 2 + 1,
                    np.where(t < 2, a * (t ** 3 - 5 * t ** 2 + 8 * t - 4), 0.0))


def _np_resize_mat():
    scale = _HR / _LR
    x = (np.arange(_HR) + 0.5) / scale - 0.5
    j = np.arange(_LR)
    w = _np_keys_cubic(x[:, None] - j[None, :])
    return (w / w.sum(axis=1, keepdims=True)).astype(np.float32)


def _np_pool_mat():
    A = np.zeros((_HR // 4, _HR), np.float32)
    for i in range(_HR // 4):
        A[i, 4 * i:4 * i + 4] = 0.25
    return A


_G_NP = _np_blur_mat()
_R_NP = _np_resize_mat()
_RB_NP = (_G_NP @ _R_NP).astype(np.float32)
_AH_NP = _np_pool_mat()

_PREC = lax.Precision.DEFAULT


def _dot(a, b):
    return lax.dot_general(a, b, (((1,), (0,)), ((), ())),
                           precision=_PREC, preferred_element_type=jnp.float32)


def _dot_t(a, b):
    return lax.dot_general(a, b, (((1,), (1,)), ((), ())),
                           precision=_PREC, preferred_element_type=jnp.float32)


def _prep_body(xhr_ref, xlr_ref, g_ref, r_ref, rb_ref, ah_ref,
               base_ref, hf_ref, hfp_ref, bhfp_ref):
    G = g_ref[...]
    R = r_ref[...]
    RB = rb_ref[...]
    AH = ah_ref[...]
    for c in range(_C):
        xc = xhr_ref[c]
        xl = xlr_ref[c]
        base = _dot_t(_dot(R, xl), R)
        bb = _dot_t(_dot(RB, xl), RB)
        bhf = base - bb
        blur = _dot_t(_dot(G, xc), G)
        hf = xc - blur
        base_ref[c] = base
        hf_ref[c] = hf
        hfp_ref[c] = _dot_t(_dot(AH, hf), AH)
        bhfp_ref[c] = _dot_t(_dot(AH, bhf), AH)


def _prep(x_hr, x_lr):
    return pl.pallas_call(
        _prep_body,
        out_shape=(
            jax.ShapeDtypeStruct((_C, _HR, _HR), jnp.float32),
            jax.ShapeDtypeStruct((_C, _HR, _HR), jnp.float32),
            jax.ShapeDtypeStruct((_C, _LR, _LR), jnp.float32),
            jax.ShapeDtypeStruct((_C, _LR, _LR), jnp.float32),
        ),
    )(x_hr, x_lr, jnp.asarray(_G_NP), jnp.asarray(_R_NP),
      jnp.asarray(_RB_NP), jnp.asarray(_AH_NP))


def _sc_patchify(hf2d, hfp2d, bhfp2d):
    mesh = plsc.VectorSubcoreMesh(core_axis_name="c", subcore_axis_name="s")

    @functools.partial(
        pl.kernel, mesh=mesh,
        out_type=(jax.ShapeDtypeStruct((_N * _N,), jnp.float32),
                  jax.ShapeDtypeStruct((_N * _DESC,), jnp.float32),
                  jax.ShapeDtypeStruct((_N * _DESC,), jnp.float32)),
        compiler_params=pltpu.CompilerParams(needs_layout_passes=False),
        scratch_types=[
            pltpu.VMEM((48, 512), jnp.float32),
            pltpu.VMEM((24, 128), jnp.float32),
            pltpu.VMEM((24, 128), jnp.float32),
            pltpu.VMEM((32 * _N,), jnp.float32),
            pltpu.VMEM((32 * _DESC,), jnp.float32),
            pltpu.VMEM((32 * _DESC,), jnp.float32),
            pltpu.SemaphoreType.DMA,
        ],
    )
    def k(hf_hbm, hfp_hbm, bhfp_hbm, hff_hbm, kp_hbm, qp_hbm,
          bandv, pbandv, qbandv, hstage, kstage, qstage, sem):
        wid = lax.axis_index("s") * 2 + lax.axis_index("c")
        iota = lax.iota(jnp.int32, 16)

        cps = []
        for c in range(_C):
            cps.append(pltpu.make_async_copy(
                hf_hbm.at[pl.ds(c * _HR + 16 * wid, 16), :],
                bandv.at[pl.ds(c * 16, 16), :], sem))
            cps.append(pltpu.make_async_copy(
                hfp_hbm.at[pl.ds(c * _LR + 8 * (wid // 2), 8), :],
                pbandv.at[pl.ds(c * 8, 8), :], sem))
            cps.append(pltpu.make_async_copy(
                bhfp_hbm.at[pl.ds(c * _LR + 8 * (wid // 2), 8), :],
                qbandv.at[pl.ds(c * 8, 8), :], sem))
        for cp in cps:
            cp.start()
        for cp in cps:
            cp.wait()

        @pl.loop(0, 48)
        def _(ci):
            for tw in range(32):
                hstage[pl.ds(tw * _N + 16 * ci, 16)] = (
                    bandv[ci, pl.ds(16 * tw, 16)])

        off = 4 * (wid % 2)
        for g in range(_C):
            rvec = g * 8 + off + iota // 4
            for tw in range(32):
                cvec = 4 * tw + (iota % 4)
                kstage[pl.ds(tw * _DESC + 16 * g, 16)] = (
                    plsc.load_gather(pbandv, [rvec, cvec]))
                qstage[pl.ds(tw * _DESC + 16 * g, 16)] = (
                    plsc.load_gather(qbandv, [rvec, cvec]))

        ocps = [
            pltpu.make_async_copy(
                hstage, hff_hbm.at[pl.ds(wid * 32 * _N, 32 * _N)], sem),
            pltpu.make_async_copy(
                kstage, kp_hbm.at[pl.ds(wid * 32 * _DESC, 32 * _DESC)], sem),
            pltpu.make_async_copy(
                qstage, qp_hbm.at[pl.ds(wid * 32 * _DESC, 32 * _DESC)], sem),
        ]
        for cp in ocps:
            cp.start()
        for cp in ocps:
            cp.wait()

    return k(hf2d, hfp2d, bhfp2d)


def _sc_fold(resc, base2d):
    mesh = plsc.VectorSubcoreMesh(core_axis_name="c", subcore_axis_name="s")

    @functools.partial(
        pl.kernel, mesh=mesh,
        out_type=jax.ShapeDtypeStruct((_C * _HR, _HR), jnp.float32),
        compiler_params=pltpu.CompilerParams(needs_layout_passes=False),
        scratch_types=[
            pltpu.VMEM((32, _FLAT), jnp.float32),
            pltpu.VMEM((48, 512), jnp.float32),
            pltpu.VMEM((48, 512), jnp.float32),
            pltpu.SemaphoreType.DMA,
        ],
    )
    def k(resc_hbm, base_hbm, out_hbm, rv, bv, ov, sem):
        wid = lax.axis_index("s") * 2 + lax.axis_index("c")
        cps = [pltpu.make_async_copy(
            resc_hbm.at[pl.ds(wid * 32, 32), :], rv, sem)]
        for c in range(_C):
            cps.append(pltpu.make_async_copy(
                base_hbm.at[pl.ds(c * _HR + 16 * wid, 16), :],
                bv.at[pl.ds(c * 16, 16), :], sem))
        for cp in cps:
            cp.start()
        for cp in cps:
            cp.wait()

        @pl.loop(0, 48)
        def _(ci):
            for tw in range(32):
                ov[ci, pl.ds(16 * tw, 16)] = (
                    bv[ci, pl.ds(16 * tw, 16)] +
                    rv[tw, pl.ds(16 * ci, 16)])

        ocps = []
        for c in range(_C):
            ocps.append(pltpu.make_async_copy(
                ov.at[pl.ds(c * 16, 16), :],
                out_hbm.at[pl.ds(c * _HR + 16 * wid, 16), :], sem))
        for cp in ocps:
            cp.start()
        for cp in ocps:
            cp.wait()

    return k(resc, base2d)


def _sc_topk(attn_flat):
    rpw = _N // _NWRK
    RIL = 4

    mesh = plsc.VectorSubcoreMesh(core_axis_name="c", subcore_axis_name="s")

    @functools.partial(
        pl.kernel, mesh=mesh,
        out_type=(jax.ShapeDtypeStruct((_N * _TOPK,), jnp.float32),
                  jax.ShapeDtypeStruct((_N * _TOPK,), jnp.int32)),
        compiler_params=pltpu.CompilerParams(needs_layout_passes=False),
        scratch_types=[
            pltpu.VMEM((rpw * _N,), jnp.float32),
            pltpu.VMEM((rpw * _TOPK,), jnp.float32),
            pltpu.VMEM((rpw * _TOPK,), jnp.int32),
        ],
    )
    def k(attn_hbm, prior_hbm, idx_hbm, rows, pv, iv):
        wid = lax.axis_index("s") * 2 + lax.axis_index("c")
        pltpu.sync_copy(attn_hbm.at[pl.ds(wid * rpw * _N, rpw * _N)], rows)
        iota = lax.iota(jnp.int32, 16)

        def row_cm(rbase):
            cms = []
            for j in range(4):
                cm = rows[pl.ds(rbase + j * 256, 16)]
                for i in range(1, 16):
                    cm = jnp.maximum(cm, rows[pl.ds(rbase + j * 256 + i * 16, 16)])
                cms.append(cm)
            return cms

        def extract(rbase, cms):
            cm0, cm1, cm2, cm3 = cms
            t = jnp.maximum(jnp.maximum(cm0, cm1), jnp.maximum(cm2, cm3))
            M = jnp.max(t)
            Ms = jnp.full((16,), M)
            eqs = [cm == Ms for cm in cms]
            ps = [plsc.all_reduce_population_count(e) for e in eqs]
            fs = [plsc.all_reduce_ffs(e) for e in eqs]
            jsel = jnp.full((16,), 3, jnp.int32)
            lsel = fs[3]
            for j in (2, 1, 0):
                hit = ps[j] > 0
                jsel = jnp.where(hit, j, jsel)
                lsel = jnp.where(hit, fs[j], lsel)
            colbase = jsel * 256 + lsel
            cvals = plsc.load_gather(rows, [rbase + colbase + iota * 16])
            eqc = cvals == Ms
            fi = plsc.all_reduce_ffs(eqc)
            col = colbase + fi * 16
            lane0 = iota == 0
            neg1 = jnp.full((16,), -1.0)
            plsc.store_scatter(rows, [rbase + col], neg1, mask=lane0)
            cv2 = jnp.where(iota == fi, neg1, cvals)
            nms = jnp.full((16,), jnp.max(cv2))
            new = []
            for j, cm in enumerate(cms):
                new.append(jnp.where((jsel == j) & (iota == lsel), nms, cm))
            return col, Ms, new

        def grp_body(g, _):
            rlocs = [g * RIL + u for u in range(RIL)]
            rbases = [rl * _N for rl in rlocs]
            init = tuple(x for rb in rbases for x in row_cm(rb))

            lane0 = iota == 0

            def step(kk, carry):
                out = []
                for u in range(RIL):
                    col, Ms, ncms = extract(rbases[u], list(carry[4 * u:4 * u + 4]))
                    addr = jnp.full((16,), rlocs[u] * _TOPK, jnp.int32) + kk
                    plsc.store_scatter(pv, [addr], Ms, mask=lane0)
                    plsc.store_scatter(iv, [addr], col, mask=lane0)
                    out += ncms
                return tuple(out)

            lax.fori_loop(0, _TOPK, step, init)
            return 0

        lax.fori_loop(0, rpw // RIL, grp_body, 0)
        pltpu.sync_copy(pv, prior_hbm.at[pl.ds(wid * rpw * _TOPK, rpw * _TOPK)])
        pltpu.sync_copy(iv, idx_hbm.at[pl.ds(wid * rpw * _TOPK, rpw * _TOPK)])

    return k(attn_flat)


def _ktable_body(kp_ref, wk_ref, bk_ref, w1_ref, out_ref):
    kge = _dot(kp_ref[...], wk_ref[...]) + bk_ref[...]
    w1 = w1_ref[...]
    w1kd = w1[_EMB:2 * _EMB] - w1[2 * _EMB:3 * _EMB]
    out_ref[:, 0:_EMB] = kge
    out_ref[:, _EMB:2 * _EMB] = _dot(kge, w1kd)


def _ktable(kp_flat, Wk, bk, W1):
    return pl.pallas_call(
        _ktable_body,
        out_shape=jax.ShapeDtypeStruct((_N, 2 * _EMB), jnp.float32),
    )(kp_flat, Wk, bk.reshape(1, _EMB), W1)


def _sc_gather(table, idx_flat):
    bpw = (_N * _TOPK) // _NWRK
    rows_per_w = bpw // 128

    mesh = plsc.VectorSubcoreMesh(core_axis_name="c", subcore_axis_name="s")

    nbuf = 4

    @functools.partial(
        pl.kernel, mesh=mesh,
        out_type=jax.ShapeDtypeStruct((_N * _TOPK, 2 * _EMB), jnp.float32),
        scratch_types=(
            [pltpu.VMEM((bpw,), jnp.int32)] +
            [pltpu.VMEM((128, 2 * _EMB), jnp.float32)] * nbuf +
            [pltpu.SemaphoreType.DMA] * nbuf +
            [pltpu.SemaphoreType.DMA] * nbuf
        ),
    )
    def k(table_hbm, idx_hbm, out_hbm, idx_v, *rest):
        bufs = rest[:nbuf]
        sems = rest[nbuf:2 * nbuf]
        osems = rest[2 * nbuf:]
        wid = lax.axis_index("s") * 2 + lax.axis_index("c")
        pltpu.sync_copy(idx_hbm.at[pl.ds(wid * bpw, bpw)], idx_v)

        def chunk(j):
            return table_hbm.at[idx_v.at[pl.ds(j * 128, 128)]]

        def out_at(j):
            return out_hbm.at[pl.ds(wid * bpw + j * 128, 128)]

        for j in range(nbuf - 1):
            pltpu.async_copy(chunk(j), bufs[j], sems[j])
        for j in range(rows_per_w):
            b = j % nbuf
            jn = j + nbuf - 1
            if jn < rows_per_w:
                bn = jn % nbuf
                if jn - nbuf >= 0:
                    pltpu.make_async_copy(bufs[bn], out_at(jn - nbuf),
                                          osems[bn]).wait()
                pltpu.async_copy(chunk(jn), bufs[bn], sems[bn])
            pltpu.make_async_copy(chunk(j), bufs[b], sems[b]).wait()
            pltpu.async_copy(bufs[b], out_at(j), osems[b])
        for j in range(rows_per_w - nbuf, rows_per_w):
            b = j % nbuf
            pltpu.make_async_copy(bufs[b], out_at(j), osems[b]).wait()

    return k(table, idx_flat)


def _sc_scatter(w_flat, idx_flat, zrows):
    rpw = _N // _NWRK
    vpw = rpw * _TOPK

    mesh = plsc.VectorSubcoreMesh(core_axis_name="c", subcore_axis_name="s")

    @functools.partial(
        pl.kernel, mesh=mesh,
        out_type=jax.ShapeDtypeStruct((_N * _N,), jnp.float32),
        compiler_params=pltpu.CompilerParams(needs_layout_passes=False),
        scratch_types=[
            pltpu.VMEM((vpw,), jnp.int32),
            pltpu.VMEM((vpw,), jnp.float32),
            pltpu.VMEM((rpw * _N,), jnp.float32),
        ],
    )
    def k(w_hbm, idx_hbm, z_hbm, s_hbm, idx_v, w_v, srow):
        wid = lax.axis_index("s") * 2 + lax.axis_index("c")
        pltpu.sync_copy(z_hbm, srow)
        pltpu.sync_copy(idx_hbm.at[pl.ds(wid * vpw, vpw)], idx_v)
        pltpu.sync_copy(w_hbm.at[pl.ds(wid * vpw, vpw)], w_v)
        for r in range(rpw):
            roff = jnp.full((16,), r * _N, jnp.int32)
            for h in range(_TOPK // 16):
                cols = idx_v[pl.ds(r * _TOPK + h * 16, 16)]
                vals = w_v[pl.ds(r * _TOPK + h * 16, 16)]
                plsc.store_scatter(srow, [cols + roff], vals)
        pltpu.sync_copy(srow, s_hbm.at[pl.ds(wid * rpw * _N, rpw * _N)])

    return k(w_flat, idx_flat, zrows)


def _mlp_body(qp_ref, kg_ref, prior_ref, wq_ref, bq_ref,
              w1_ref, b1_ref, w2_ref, b2_ref, w_ref):
    blk = w_ref.shape[0]
    prior = prior_ref[...]
    qe = _dot(qp_ref[...], wq_ref[...]) + bq_ref[...]
    kge = kg_ref[:, 0:_EMB]
    Bk = kg_ref[:, _EMB:2 * _EMB]
    w1 = w1_ref[...]
    w1q = w1[0:_EMB]
    w1d = w1[2 * _EMB:3 * _EMB]
    w1m = w1[3 * _EMB:4 * _EMB]
    w1p = w1[4 * _EMB:4 * _EMB + 1]
    A = _dot(qe, w1q + w1d) + b1_ref[...]
    kge3 = kge.reshape(blk, _TOPK, _EMB)
    qk = (qe.reshape(blk, 1, _EMB) * kge3).reshape(blk * _TOPK, _EMB)
    Cm = _dot(qk, w1m)
    h = (Bk + Cm).reshape(blk, _TOPK, _EMB)
    h = h + A.reshape(blk, 1, _EMB)
    h = h + prior.reshape(blk, _TOPK, 1) * w1p.reshape(1, 1, _EMB)
    h = jnp.maximum(h, 0.0)
    res = jnp.sum(h * w2_ref[...].reshape(1, 1, _EMB), axis=-1) + b2_ref[0, 0]
    logits = jnp.log(jnp.maximum(prior, 1e-8)) + res
    m = jnp.max(logits, axis=-1, keepdims=True)
    e = jnp.exp(logits - m)
    w_ref[...] = e / jnp.sum(e, axis=-1, keepdims=True)


def _mlp(qp, kg, prior, Wq, bq, W1, b1, W2, b2):
    blk = 256
    grid = _N // blk
    return pl.pallas_call(
        _mlp_body,
        grid=(grid,),
        in_specs=[
            pl.BlockSpec((blk, _DESC), lambda i: (i, 0)),
            pl.BlockSpec((blk * _TOPK, 2 * _EMB), lambda i: (i, 0)),
            pl.BlockSpec((blk, _TOPK), lambda i: (i, 0)),
            pl.BlockSpec((_DESC, _EMB), lambda i: (0, 0)),
            pl.BlockSpec((1, _EMB), lambda i: (0, 0)),
            pl.BlockSpec((4 * _EMB + 1, _EMB), lambda i: (0, 0)),
            pl.BlockSpec((1, _EMB), lambda i: (0, 0)),
            pl.BlockSpec((_EMB, 1), lambda i: (0, 0)),
            pl.BlockSpec((1, 1), lambda i: (0, 0)),
        ],
        out_specs=pl.BlockSpec((blk, _TOPK), lambda i: (i, 0)),
        out_shape=jax.ShapeDtypeStruct((_N, _TOPK), jnp.float32),
    )(qp, kg, prior, Wq, bq.reshape(1, _EMB),
      W1, b1.reshape(1, _EMB), W2, b2.reshape(1, 1))


def _combine_body(s_ref, hf_ref, out_ref):
    blk = out_ref.shape[0]
    s = s_ref[...].reshape(blk, _N)
    hf = hf_ref[...].reshape(_N, _N)
    out_ref[...] = _dot(s, hf)[:, 0:_FLAT]


def _combine(S_flat, hf_flat):
    blk = 256
    return pl.pallas_call(
        _combine_body,
        grid=(_N // blk,),
        in_specs=[
            pl.BlockSpec((blk * _N,), lambda i: (i,)),
            pl.BlockSpec((_N * _N,), lambda i: (0,)),
        ],
        out_specs=pl.BlockSpec((blk, _FLAT), lambda i: (i, 0)),
        out_shape=jax.ShapeDtypeStruct((_N, _FLAT), jnp.float32),
    )(S_flat, hf_flat)


def _img_to_flat(img):
    x = img.reshape(_C, _NH, _PS, _NH, _PS)
    return x.transpose(1, 3, 0, 2, 4).reshape(_N, _FLAT)


def _pooled_to_flat(img):
    x = img.reshape(_C, _NH, _POOL, _NH, _POOL)
    return x.transpose(1, 3, 0, 2, 4).reshape(_N, _DESC)


def _flat_to_img(pf):
    x = pf.reshape(_NH, _NH, _C, _PS, _PS)
    return x.transpose(2, 0, 3, 1, 4).reshape(1, _C, _HR, _HR)


def kernel(x_hr, x_lr_inpainted, attn_map, Wk, bk, Wq, bq, W1, b1, W2, b2):
    base_img, hf_img, hfp, bhfp = _prep(x_hr[0], x_lr_inpainted[0])

    hf_flat, kp_flat, qp_flat = _sc_patchify(
        hf_img.reshape(_C * _HR, _HR),
        hfp.reshape(_C * _LR, _LR),
        bhfp.reshape(_C * _LR, _LR))

    pr_flat, ix_flat = _sc_topk(attn_map.reshape(-1))

    ktab = _ktable(kp_flat.reshape(_N, _DESC), Wk, bk, W1)
    kg = _sc_gather(ktab, ix_flat)
    w = _mlp(qp_flat.reshape(_N, _DESC), kg, pr_flat.reshape(_N, _TOPK),
             Wq, bq, W1, b1, W2, b2)

    S_flat = _sc_scatter(w.reshape(-1), ix_flat,
                         jnp.zeros((_N * _TOPK,), jnp.float32))
    resc = _combine(S_flat, hf_flat)
    final2d = _sc_fold(resc, base_img.reshape(_C * _HR, _HR))
    return final2d.reshape(1, _C, _HR, _HR)


# speedup vs baseline: 1.1279x; 1.0008x over previous
import functools

import jax
import jax.numpy as jnp
import numpy as np
from jax import lax
from jax.experimental import pallas as pl
from jax.experimental.pallas import tpu as pltpu
from jax.experimental.pallas import tpu_sc as plsc

_C = 3
_HR = 512
_LR = 128
_PS = 16
_NH = 32
_N = 1024
_POOL = 4
_DESC = 48
_EMB = 64
_TOPK = 32
_FLAT = _C * _PS * _PS

_NWRK = 32


def _np_gauss1d():
    x = np.arange(5, dtype=np.float64) - 2.0
    k1 = np.exp(-(x ** 2) / 2.0)
    return (k1 / k1.sum()).astype(np.float32)


def _np_blur_mat():
    k1 = _np_gauss1d()
    G = np.zeros((_HR, _HR), np.float32)
    for t in range(5):
        off = t - 2
        for i in range(_HR):
            j = i + off
            if 0 <= j < _HR:
                G[i, j] += k1[t]
    return G


def _np_keys_cubic(t, a=-0.5):
    t = np.abs(t)
    return np.where(t <= 1, (a + 2) * t ** 3 - (a + 3) * t ** 2 + 1,
                    np.where(t < 2, a * (t ** 3 - 5 * t ** 2 + 8 * t - 4), 0.0))


def _np_resize_mat():
    scale = _HR / _LR
    x = (np.arange(_HR) + 0.5) / scale - 0.5
    j = np.arange(_LR)
    w = _np_keys_cubic(x[:, None] - j[None, :])
    return (w / w.sum(axis=1, keepdims=True)).astype(np.float32)


def _np_pool_mat():
    A = np.zeros((_HR // 4, _HR), np.float32)
    for i in range(_HR // 4):
        A[i, 4 * i:4 * i + 4] = 0.25
    return A


_G_NP = _np_blur_mat()
_R_NP = _np_resize_mat()
_RB_NP = (_G_NP @ _R_NP).astype(np.float32)
_AH_NP = _np_pool_mat()

_PREC = lax.Precision.DEFAULT


def _dot(a, b):
    return lax.dot_general(a, b, (((1,), (0,)), ((), ())),
                           precision=_PREC, preferred_element_type=jnp.float32)


def _dot_t(a, b):
    return lax.dot_general(a, b, (((1,), (1,)), ((), ())),
                           precision=_PREC, preferred_element_type=jnp.float32)


def _prep_body(xhr_ref, xlr_ref, g_ref, r_ref, rb_ref, ah_ref,
               base_ref, hf_ref, hfp_ref, bhfp_ref):
    G = g_ref[...]
    R = r_ref[...]
    RB = rb_ref[...]
    AH = ah_ref[...]
    for c in range(_C):
        xc = xhr_ref[c]
        xl = xlr_ref[c]
        base = _dot_t(_dot(R, xl), R)
        bb = _dot_t(_dot(RB, xl), RB)
        bhf = base - bb
        blur = _dot_t(_dot(G, xc), G)
        hf = xc - blur
        base_ref[c] = base
        hf_ref[c] = hf
        hfp_ref[c] = _dot_t(_dot(AH, hf), AH)
        bhfp_ref[c] = _dot_t(_dot(AH, bhf), AH)


def _prep(x_hr, x_lr):
    return pl.pallas_call(
        _prep_body,
        out_shape=(
            jax.ShapeDtypeStruct((_C, _HR, _HR), jnp.float32),
            jax.ShapeDtypeStruct((_C, _HR, _HR), jnp.float32),
            jax.ShapeDtypeStruct((_C, _LR, _LR), jnp.float32),
            jax.ShapeDtypeStruct((_C, _LR, _LR), jnp.float32),
        ),
    )(x_hr, x_lr, jnp.asarray(_G_NP), jnp.asarray(_R_NP),
      jnp.asarray(_RB_NP), jnp.asarray(_AH_NP))


def _sc_patchify(hf2d, hfp2d, bhfp2d):
    mesh = plsc.VectorSubcoreMesh(core_axis_name="c", subcore_axis_name="s")

    @functools.partial(
        pl.kernel, mesh=mesh,
        out_type=(jax.ShapeDtypeStruct((_N * _N,), jnp.float32),
                  jax.ShapeDtypeStruct((_N * _DESC,), jnp.float32),
                  jax.ShapeDtypeStruct((_N * _DESC,), jnp.float32)),
        compiler_params=pltpu.CompilerParams(needs_layout_passes=False),
        scratch_types=[
            pltpu.VMEM((48, 512), jnp.float32),
            pltpu.VMEM((24, 128), jnp.float32),
            pltpu.VMEM((24, 128), jnp.float32),
            pltpu.VMEM((32 * _N,), jnp.float32),
            pltpu.VMEM((32 * _DESC,), jnp.float32),
            pltpu.VMEM((32 * _DESC,), jnp.float32),
            pltpu.SemaphoreType.DMA,
        ],
    )
    def k(hf_hbm, hfp_hbm, bhfp_hbm, hff_hbm, kp_hbm, qp_hbm,
          bandv, pbandv, qbandv, hstage, kstage, qstage, sem):
        wid = lax.axis_index("s") * 2 + lax.axis_index("c")
        iota = lax.iota(jnp.int32, 16)

        cps = []
        for c in range(_C):
            cps.append(pltpu.make_async_copy(
                hf_hbm.at[pl.ds(c * _HR + 16 * wid, 16), :],
                bandv.at[pl.ds(c * 16, 16), :], sem))
            cps.append(pltpu.make_async_copy(
                hfp_hbm.at[pl.ds(c * _LR + 8 * (wid // 2), 8), :],
                pbandv.at[pl.ds(c * 8, 8), :], sem))
            cps.append(pltpu.make_async_copy(
                bhfp_hbm.at[pl.ds(c * _LR + 8 * (wid // 2), 8), :],
                qbandv.at[pl.ds(c * 8, 8), :], sem))
        for cp in cps:
            cp.start()
        for cp in cps:
            cp.wait()

        @pl.loop(0, 48)
        def _(ci):
            for tw in range(32):
                hstage[pl.ds(tw * _N + 16 * ci, 16)] = (
                    bandv[ci, pl.ds(16 * tw, 16)])

        off = 4 * (wid % 2)
        for g in range(_C):
            rvec = g * 8 + off + iota // 4
            for tw in range(32):
                cvec = 4 * tw + (iota % 4)
                kstage[pl.ds(tw * _DESC + 16 * g, 16)] = (
                    plsc.load_gather(pbandv, [rvec, cvec]))
                qstage[pl.ds(tw * _DESC + 16 * g, 16)] = (
                    plsc.load_gather(qbandv, [rvec, cvec]))

        ocps = [
            pltpu.make_async_copy(
                hstage, hff_hbm.at[pl.ds(wid * 32 * _N, 32 * _N)], sem),
            pltpu.make_async_copy(
                kstage, kp_hbm.at[pl.ds(wid * 32 * _DESC, 32 * _DESC)], sem),
            pltpu.make_async_copy(
                qstage, qp_hbm.at[pl.ds(wid * 32 * _DESC, 32 * _DESC)], sem),
        ]
        for cp in ocps:
            cp.start()
        for cp in ocps:
            cp.wait()

    return k(hf2d, hfp2d, bhfp2d)


def _sc_fold(resc, base2d):
    mesh = plsc.VectorSubcoreMesh(core_axis_name="c", subcore_axis_name="s")

    @functools.partial(
        pl.kernel, mesh=mesh,
        out_type=jax.ShapeDtypeStruct((_C * _HR, _HR), jnp.float32),
        compiler_params=pltpu.CompilerParams(needs_layout_passes=False),
        scratch_types=[
            pltpu.VMEM((32, _FLAT), jnp.float32),
            pltpu.VMEM((48, 512), jnp.float32),
            pltpu.VMEM((48, 512), jnp.float32),
            pltpu.SemaphoreType.DMA,
        ],
    )
    def k(resc_hbm, base_hbm, out_hbm, rv, bv, ov, sem):
        wid = lax.axis_index("s") * 2 + lax.axis_index("c")
        cps = [pltpu.make_async_copy(
            resc_hbm.at[pl.ds(wid * 32, 32), :], rv, sem)]
        for c in range(_C):
            cps.append(pltpu.make_async_copy(
                base_hbm.at[pl.ds(c * _HR + 16 * wid, 16), :],
                bv.at[pl.ds(c * 16, 16), :], sem))
        for cp in cps:
            cp.start()
        for cp in cps:
            cp.wait()

        @pl.loop(0, 48)
        def _(ci):
            for tw in range(32):
                ov[ci, pl.ds(16 * tw, 16)] = (
                    bv[ci, pl.ds(16 * tw, 16)] +
                    rv[tw, pl.ds(16 * ci, 16)])

        ocps = []
        for c in range(_C):
            ocps.append(pltpu.make_async_copy(
                ov.at[pl.ds(c * 16, 16), :],
                out_hbm.at[pl.ds(c * _HR + 16 * wid, 16), :], sem))
        for cp in ocps:
            cp.start()
        for cp in ocps:
            cp.wait()

    return k(resc, base2d)


def _sc_topk(attn_flat):
    rpw = _N // _NWRK
    RIL = 4

    mesh = plsc.VectorSubcoreMesh(core_axis_name="c", subcore_axis_name="s")

    @functools.partial(
        pl.kernel, mesh=mesh,
        out_type=(jax.ShapeDtypeStruct((_N * _TOPK,), jnp.float32),
                  jax.ShapeDtypeStruct((_N * _TOPK,), jnp.int32)),
        compiler_params=pltpu.CompilerParams(needs_layout_passes=False),
        scratch_types=[
            pltpu.VMEM((rpw * _N,), jnp.float32),
            pltpu.VMEM((rpw * _TOPK,), jnp.float32),
            pltpu.VMEM((rpw * _TOPK,), jnp.int32),
        ],
    )
    def k(attn_hbm, prior_hbm, idx_hbm, rows, pv, iv):
        wid = lax.axis_index("s") * 2 + lax.axis_index("c")
        pltpu.sync_copy(attn_hbm.at[pl.ds(wid * rpw * _N, rpw * _N)], rows)
        iota = lax.iota(jnp.int32, 16)

        def row_cm(rbase):
            cms = []
            for j in range(4):
                cm = rows[pl.ds(rbase + j * 256, 16)]
                for i in range(1, 16):
                    cm = jnp.maximum(cm, rows[pl.ds(rbase + j * 256 + i * 16, 16)])
                cms.append(cm)
            return cms

        def extract(rbase, cms):
            cm0, cm1, cm2, cm3 = cms
            t = jnp.maximum(jnp.maximum(cm0, cm1), jnp.maximum(cm2, cm3))
            M = jnp.max(t)
            Ms = jnp.full((16,), M)
            eqs = [cm == Ms for cm in cms]
            ps = [plsc.all_reduce_population_count(e) for e in eqs]
            fs = [plsc.all_reduce_ffs(e) for e in eqs]
            jsel = jnp.full((16,), 3, jnp.int32)
            lsel = fs[3]
            for j in (2, 1, 0):
                hit = ps[j] > 0
                jsel = jnp.where(hit, j, jsel)
                lsel = jnp.where(hit, fs[j], lsel)
            colbase = jsel * 256 + lsel
            cvals = plsc.load_gather(rows, [rbase + colbase + iota * 16])
            eqc = cvals == Ms
            fi = plsc.all_reduce_ffs(eqc)
            col = colbase + fi * 16
            lane0 = iota == 0
            neg1 = jnp.full((16,), -1.0)
            plsc.store_scatter(rows, [rbase + col], neg1, mask=lane0)
            cv2 = jnp.where(iota == fi, neg1, cvals)
            nms = jnp.full((16,), jnp.max(cv2))
            new = []
            for j, cm in enumerate(cms):
                new.append(jnp.where((jsel == j) & (iota == lsel), nms, cm))
            return col, Ms, new

        def grp_body(g, _):
            rlocs = [g * RIL + u for u in range(RIL)]
            rbases = [rl * _N for rl in rlocs]
            init = tuple(x for rb in rbases for x in row_cm(rb))

            lane0 = iota == 0

            def step(kk, carry):
                out = []
                for u in range(RIL):
                    col, Ms, ncms = extract(rbases[u], list(carry[4 * u:4 * u + 4]))
                    addr = jnp.full((16,), rlocs[u] * _TOPK, jnp.int32) + kk
                    plsc.store_scatter(pv, [addr], Ms, mask=lane0)
                    plsc.store_scatter(iv, [addr], col, mask=lane0)
                    out += ncms
                return tuple(out)

            lax.fori_loop(0, _TOPK, step, init)
            return 0

        lax.fori_loop(0, rpw // RIL, grp_body, 0)
        pltpu.sync_copy(pv, prior_hbm.at[pl.ds(wid * rpw * _TOPK, rpw * _TOPK)])
        pltpu.sync_copy(iv, idx_hbm.at[pl.ds(wid * rpw * _TOPK, rpw * _TOPK)])

    return k(attn_flat)


def _ktable_body(kp_ref, wk_ref, bk_ref, w1_ref, out_ref):
    kge = _dot(kp_ref[...], wk_ref[...]) + bk_ref[...]
    w1 = w1_ref[...]
    w1kd = w1[_EMB:2 * _EMB] - w1[2 * _EMB:3 * _EMB]
    out_ref[:, 0:_EMB] = kge
    out_ref[:, _EMB:2 * _EMB] = _dot(kge, w1kd)


def _ktable(kp_flat, Wk, bk, W1):
    return pl.pallas_call(
        _ktable_body,
        out_shape=jax.ShapeDtypeStruct((_N, 2 * _EMB), jnp.float32),
    )(kp_flat, Wk, bk.reshape(1, _EMB), W1)


def _sc_gather(table, idx_flat):
    bpw = (_N * _TOPK) // _NWRK
    rows_per_w = bpw // 128

    mesh = plsc.VectorSubcoreMesh(core_axis_name="c", subcore_axis_name="s")

    nbuf = 4

    @functools.partial(
        pl.kernel, mesh=mesh,
        out_type=jax.ShapeDtypeStruct((_N * _TOPK, 2 * _EMB), jnp.float32),
        scratch_types=(
            [pltpu.VMEM((bpw,), jnp.int32)] +
            [pltpu.VMEM((128, 2 * _EMB), jnp.float32)] * nbuf +
            [pltpu.SemaphoreType.DMA] * nbuf +
            [pltpu.SemaphoreType.DMA] * nbuf
        ),
    )
    def k(table_hbm, idx_hbm, out_hbm, idx_v, *rest):
        bufs = rest[:nbuf]
        sems = rest[nbuf:2 * nbuf]
        osems = rest[2 * nbuf:]
        wid = lax.axis_index("s") * 2 + lax.axis_index("c")
        pltpu.sync_copy(idx_hbm.at[pl.ds(wid * bpw, bpw)], idx_v)

        def chunk(j):
            return table_hbm.at[idx_v.at[pl.ds(j * 128, 128)]]

        def out_at(j):
            return out_hbm.at[pl.ds(wid * bpw + j * 128, 128)]

        for j in range(nbuf - 1):
            pltpu.async_copy(chunk(j), bufs[j], sems[j])
        for j in range(rows_per_w):
            b = j % nbuf
            jn = j + nbuf - 1
            if jn < rows_per_w:
                bn = jn % nbuf
                if jn - nbuf >= 0:
                    pltpu.make_async_copy(bufs[bn], out_at(jn - nbuf),
                                          osems[bn]).wait()
                pltpu.async_copy(chunk(jn), bufs[bn], sems[bn])
            pltpu.make_async_copy(chunk(j), bufs[b], sems[b]).wait()
            pltpu.async_copy(bufs[b], out_at(j), osems[b])
        for j in range(rows_per_w - nbuf, rows_per_w):
            b = j % nbuf
            pltpu.make_async_copy(bufs[b], out_at(j), osems[b]).wait()

    return k(table, idx_flat)


def _sc_scatter(w_flat, idx_flat, zrows):
    rpw = _N // _NWRK
    vpw = rpw * _TOPK

    mesh = plsc.VectorSubcoreMesh(core_axis_name="c", subcore_axis_name="s")

    @functools.partial(
        pl.kernel, mesh=mesh,
        out_type=jax.ShapeDtypeStruct((_N * _N,), jnp.float32),
        compiler_params=pltpu.CompilerParams(needs_layout_passes=False),
        scratch_types=[
            pltpu.VMEM((vpw,), jnp.int32),
            pltpu.VMEM((vpw,), jnp.float32),
            pltpu.VMEM((rpw * _N,), jnp.float32),
        ],
    )
    def k(w_hbm, idx_hbm, z_hbm, s_hbm, idx_v, w_v, srow):
        wid = lax.axis_index("s") * 2 + lax.axis_index("c")
        pltpu.sync_copy(z_hbm, srow)
        pltpu.sync_copy(idx_hbm.at[pl.ds(wid * vpw, vpw)], idx_v)
        pltpu.sync_copy(w_hbm.at[pl.ds(wid * vpw, vpw)], w_v)
        for r in range(rpw):
            roff = jnp.full((16,), r * _N, jnp.int32)
            for h in range(_TOPK // 16):
                cols = idx_v[pl.ds(r * _TOPK + h * 16, 16)]
                vals = w_v[pl.ds(r * _TOPK + h * 16, 16)]
                plsc.store_scatter(srow, [cols + roff], vals)
        pltpu.sync_copy(srow, s_hbm.at[pl.ds(wid * rpw * _N, rpw * _N)])

    return k(w_flat, idx_flat, zrows)


def _mlp_body(qp_ref, kg_ref, prior_ref, wq_ref, bq_ref,
              w1_ref, b1_ref, w2_ref, b2_ref, w_ref):
    blk = w_ref.shape[0]
    prior = prior_ref[...]
    qe = _dot(qp_ref[...], wq_ref[...]) + bq_ref[...]
    kge = kg_ref[:, 0:_EMB]
    Bk = kg_ref[:, _EMB:2 * _EMB]
    w1 = w1_ref[...]
    w1q = w1[0:_EMB]
    w1d = w1[2 * _EMB:3 * _EMB]
    w1m = w1[3 * _EMB:4 * _EMB]
    w1p = w1[4 * _EMB:4 * _EMB + 1]
    A = _dot(qe, w1q + w1d) + b1_ref[...]
    kge3 = kge.reshape(blk, _TOPK, _EMB)
    qk = (qe.reshape(blk, 1, _EMB) * kge3).reshape(blk * _TOPK, _EMB)
    Cm = _dot(qk, w1m)
    h = (Bk + Cm).reshape(blk, _TOPK, _EMB)
    h = h + A.reshape(blk, 1, _EMB)
    h = h + prior.reshape(blk, _TOPK, 1) * w1p.reshape(1, 1, _EMB)
    h = jnp.maximum(h, 0.0)
    res = jnp.sum(h * w2_ref[...].reshape(1, 1, _EMB), axis=-1) + b2_ref[0, 0]
    logits = jnp.log(jnp.maximum(prior, 1e-8)) + res
    m = jnp.max(logits, axis=-1, keepdims=True)
    e = jnp.exp(logits - m)
    w_ref[...] = e / jnp.sum(e, axis=-1, keepdims=True)


def _mlp(qp, kg, prior, Wq, bq, W1, b1, W2, b2):
    blk = 256
    grid = _N // blk
    return pl.pallas_call(
        _mlp_body,
        grid=(grid,),
        in_specs=[
            pl.BlockSpec((blk, _DESC), lambda i: (i, 0)),
            pl.BlockSpec((blk * _TOPK, 2 * _EMB), lambda i: (i, 0)),
            pl.BlockSpec((blk, _TOPK), lambda i: (i, 0)),
            pl.BlockSpec((_DESC, _EMB), lambda i: (0, 0)),
            pl.BlockSpec((1, _EMB), lambda i: (0, 0)),
            pl.BlockSpec((4 * _EMB + 1, _EMB), lambda i: (0, 0)),
            pl.BlockSpec((1, _EMB), lambda i: (0, 0)),
            pl.BlockSpec((_EMB, 1), lambda i: (0, 0)),
            pl.BlockSpec((1, 1), lambda i: (0, 0)),
        ],
        out_specs=pl.BlockSpec((blk, _TOPK), lambda i: (i, 0)),
        out_shape=jax.ShapeDtypeStruct((_N, _TOPK), jnp.float32),
    )(qp, kg, prior, Wq, bq.reshape(1, _EMB),
      W1, b1.reshape(1, _EMB), W2, b2.reshape(1, 1))


def _combine_body(s_ref, hf_ref, out_ref):
    blk = out_ref.shape[0]
    s = s_ref[...].reshape(blk, _N)
    hf = hf_ref[...].reshape(_N, _N)
    out_ref[...] = _dot(s, hf)[:, 0:_FLAT]


def _combine(S_flat, hf_flat):
    blk = 256
    return pl.pallas_call(
        _combine_body,
        grid=(_N // blk,),
        in_specs=[
            pl.BlockSpec((blk * _N,), lambda i: (i,)),
            pl.BlockSpec((_N * _N,), lambda i: (0,)),
        ],
        out_specs=pl.BlockSpec((blk, _FLAT), lambda i: (i, 0)),
        out_shape=jax.ShapeDtypeStruct((_N, _FLAT), jnp.float32),
    )(S_flat, hf_flat)


def _img_to_flat(img):
    x = img.reshape(_C, _NH, _PS, _NH, _PS)
    return x.transpose(1, 3, 0, 2, 4).reshape(_N, _FLAT)


def _pooled_to_flat(img):
    x = img.reshape(_C, _NH, _POOL, _NH, _POOL)
    return x.transpose(1, 3, 0, 2, 4).reshape(_N, _DESC)


def _flat_to_img(pf):
    x = pf.reshape(_NH, _NH, _C, _PS, _PS)
    return x.transpose(2, 0, 3, 1, 4).reshape(1, _C, _HR, _HR)


def kernel(x_hr, x_lr_inpainted, attn_map, Wk, bk, Wq, bq, W1, b1, W2, b2):
    pr_flat, ix_flat = _sc_topk(attn_map.reshape(-1))

    base_img, hf_img, hfp, bhfp = _prep(x_hr[0], x_lr_inpainted[0])

    hf_flat, kp_flat, qp_flat = _sc_patchify(
        hf_img.reshape(_C * _HR, _HR),
        hfp.reshape(_C * _LR, _LR),
        bhfp.reshape(_C * _LR, _LR))

    ktab = _ktable(kp_flat.reshape(_N, _DESC), Wk, bk, W1)
    kg = _sc_gather(ktab, ix_flat)
    w = _mlp(qp_flat.reshape(_N, _DESC), kg, pr_flat.reshape(_N, _TOPK),
             Wq, bq, W1, b1, W2, b2)

    S_flat = _sc_scatter(w.reshape(-1), ix_flat,
                         jnp.zeros((_N * _TOPK,), jnp.float32))
    resc = _combine(S_flat, hf_flat)
    final2d = _sc_fold(resc, base_img.reshape(_C * _HR, _HR))
    return final2d.reshape(1, _C, _HR, _HR)
